# baseline (speedup 1.0000x reference)
import jax
import jax.numpy as jnp
from jax import lax
from jax.experimental import pallas as pl
from jax.experimental.pallas import tpu as pltpu
from jax.experimental.pallas import tpu_sc as plsc

N_TOKENS = 32768
D_IN = 1024
N_EXPERTS = 8
NW = 32
T_CH = 32
TG = 4
TPW = N_TOKENS // NW
N_CHUNKS = TPW // T_CH


def _sc_body(z_hbm, w_hbm, b_hbm, out_hbm,
             zbuf0, zbuf1, wbuf, bbuf, red, asm, sm, outstage, sem0, sem1):
    wid = lax.axis_index("s") * 2 + lax.axis_index("c")
    wbase = wid * TPW

    pltpu.sync_copy(w_hbm, wbuf)
    pltpu.sync_copy(b_hbm, bbuf)
    bvec = bbuf[...]

    iota = lax.iota(jnp.int32, 16)
    mask8 = iota < 8

    def z_src(c):
        return z_hbm.at[pl.ds((wbase + c * T_CH) * D_IN, T_CH * D_IN)]

    def start(c, buf, sem):
        pltpu.async_copy(z_src(c), buf, sem)

    def wait(c, buf, sem):
        pltpu.make_async_copy(z_src(c), buf, sem).wait()

    def fold_max(v):
        sm[pl.ds(0, 16)] = v
        r = jnp.maximum(v, sm[pl.ds(4, 16)])
        sm[pl.ds(0, 16)] = r
        r = jnp.maximum(r, sm[pl.ds(2, 16)])
        sm[pl.ds(0, 16)] = r
        r = jnp.maximum(r, sm[pl.ds(1, 16)])
        return r[0], r[8]

    def fold_sum(v):
        sm[pl.ds(0, 16)] = v
        r = v + sm[pl.ds(4, 16)]
        sm[pl.ds(0, 16)] = r
        r = r + sm[pl.ds(2, 16)]
        sm[pl.ds(0, 16)] = r
        r = r + sm[pl.ds(1, 16)]
        return r[0], r[8]

    def process(c, zbuf):
        def group(g, _):
            gbase = g * TG
            zoffs = [(gbase + k) * D_IN for k in range(TG)]

            def dc_body(dc, accs):
                off = dc * 16
                wv = [wbuf[pl.ds(e * D_IN + off, 16)] for e in range(N_EXPERTS)]
                out = []
                for k in range(TG):
                    zv = zbuf[pl.ds(zoffs[k] + off, 16)]
                    out.append(tuple(accs[k][e] + zv * wv[e]
                                     for e in range(N_EXPERTS)))
                return tuple(out)

            zero = jnp.zeros((16,), jnp.float32)
            init = tuple(tuple(zero for _ in range(N_EXPERTS))
                         for _ in range(TG))
            accs = lax.fori_loop(0, D_IN // 16, dc_body, init,
                                 unroll=False)

            for k in range(0, TG, 2):
                v = jnp.zeros((16,), jnp.float32)
                for kk in (k, k + 1):
                    for e in range(N_EXPERTS):
                        red[pl.ds(e * 16, 16)] = accs[kk][e]
                    for e in range(N_EXPERTS):
                        h = accs[kk][e] + red[pl.ds(e * 16 + 8, 16)]
                        red[pl.ds(e * 16, 16)] = h
                        q = h + red[pl.ds(e * 16 + 4, 16)]
                        red[pl.ds(e * 16, 16)] = q
                        r = q + red[pl.ds(e * 16 + 2, 16)]
                        s = r[0] + r[1]
                        lane = (kk - k) * 8 + e
                        v = jnp.where(iota == lane, jnp.full((16,), s), v)
                v = v + bvec
                m_a, m_b = fold_max(v)
                m = jnp.where(mask8, jnp.full((16,), m_a),
                              jnp.full((16,), m_b))
                e_v = jnp.exp(v - m)
                s_a, s_b = fold_sum(e_v)
                d = jnp.where(mask8, jnp.full((16,), s_a),
                              jnp.full((16,), s_b))
                p = e_v / d
                outstage[pl.ds((gbase + k) * N_EXPERTS, 16)] = p
            return 0

        lax.fori_loop(0, T_CH // TG, group, 0, unroll=False)
        tok0 = wbase + c * T_CH
        pltpu.sync_copy(outstage.at[pl.ds(0, T_CH * N_EXPERTS)],
                        out_hbm.at[pl.ds(tok0 * N_EXPERTS,
                                         T_CH * N_EXPERTS)])

    start(0, zbuf0, sem0)
    start(1, zbuf1, sem1)

    def chunk_pair(i, _):
        c0 = 2 * i
        wait(c0, zbuf0, sem0)
        process(c0, zbuf0)

        @pl.when(c0 + 2 < N_CHUNKS)
        def _():
            start(c0 + 2, zbuf0, sem0)

        c1 = c0 + 1
        wait(c1, zbuf1, sem1)
        process(c1, zbuf1)

        @pl.when(c1 + 2 < N_CHUNKS)
        def _():
            start(c1 + 2, zbuf1, sem1)
        return 0

    lax.fori_loop(0, N_CHUNKS // 2, chunk_pair, 0, unroll=False)


_SCRATCH = [
    pltpu.VMEM((T_CH * D_IN,), jnp.float32),
    pltpu.VMEM((T_CH * D_IN,), jnp.float32),
    pltpu.VMEM((N_EXPERTS * D_IN,), jnp.float32),
    pltpu.VMEM((16,), jnp.float32),
    pltpu.VMEM((N_EXPERTS * 16 + 16,), jnp.float32),
    pltpu.VMEM((16,), jnp.float32),
    pltpu.VMEM((32,), jnp.float32),
    pltpu.VMEM((T_CH * N_EXPERTS,), jnp.float32),
    pltpu.SemaphoreType.DMA,
    pltpu.SemaphoreType.DMA,
]


def kernel(z, W, b):
    n = z.shape[0]
    zf = z.reshape(n * D_IN)
    wf = W.reshape(N_EXPERTS * D_IN)
    b16 = jnp.concatenate([b, b])
    mesh = plsc.VectorSubcoreMesh(core_axis_name="c", subcore_axis_name="s")
    out = pl.kernel(
        _sc_body,
        out_type=jax.ShapeDtypeStruct((n * N_EXPERTS,), jnp.float32),
        mesh=mesh,
        scratch_types=_SCRATCH,
    )(zf, wf, b16)
    return out.reshape(n, N_EXPERTS)


# speedup vs baseline: 2.5160x; 2.5160x over previous
import jax
import jax.numpy as jnp
from jax import lax
from jax.experimental import pallas as pl
from jax.experimental.pallas import tpu as pltpu
from jax.experimental.pallas import tpu_sc as plsc

N_TOKENS = 32768
D_IN = 1024
N_EXPERTS = 8


BT = 2048


def _tc_body(z_ref, w_ref, b_ref, out_ref):
    z = z_ref[...]
    w = w_ref[...]
    logits = lax.dot_general(z, w, (((1,), (1,)), ((), ())),
                             preferred_element_type=jnp.float32)
    logits = logits + b_ref[...]
    m = jnp.max(logits, axis=-1, keepdims=True)
    e = jnp.exp(logits - m)
    s = jnp.sum(e, axis=-1, keepdims=True)
    out_ref[...] = e / s


def _tc_router(z, W, b2, n):
    return pl.pallas_call(
        _tc_body,
        grid=(n // BT,),
        in_specs=[
            pl.BlockSpec((BT, D_IN), lambda i: (i, 0)),
            pl.BlockSpec((N_EXPERTS, D_IN), lambda i: (0, 0)),
            pl.BlockSpec((1, N_EXPERTS), lambda i: (0, 0)),
        ],
        out_specs=pl.BlockSpec((BT, N_EXPERTS), lambda i: (i, 0)),
        out_shape=jax.ShapeDtypeStruct((n, N_EXPERTS), jnp.float32),
    )(z, W, b2)


NW = 32
T_CH = 32
TG = 4


def _sc_body_for(n_sc, base):
    tpw = n_sc // NW
    n_chunks = tpw // T_CH

    def _sc_body(z_hbm, w_hbm, b_hbm, out_hbm,
                 zbuf0, zbuf1, wbuf, bbuf, red, sm, outstage, sem0, sem1):
        wid = lax.axis_index("s") * 2 + lax.axis_index("c")
        wbase = wid * tpw

        pltpu.sync_copy(w_hbm, wbuf)
        pltpu.sync_copy(b_hbm, bbuf)
        bvec = bbuf[...]

        iota = lax.iota(jnp.int32, 16)
        mask8 = iota < 8

        def z_src(c):
            return z_hbm.at[pl.ds((base + wbase + c * T_CH) * D_IN,
                                  T_CH * D_IN)]

        def start(c, buf, sem):
            pltpu.async_copy(z_src(c), buf, sem)

        def wait(c, buf, sem):
            pltpu.make_async_copy(z_src(c), buf, sem).wait()

        def fold_max(v):
            sm[pl.ds(0, 16)] = v
            r = jnp.maximum(v, sm[pl.ds(4, 16)])
            sm[pl.ds(0, 16)] = r
            r = jnp.maximum(r, sm[pl.ds(2, 16)])
            sm[pl.ds(0, 16)] = r
            r = jnp.maximum(r, sm[pl.ds(1, 16)])
            return r[0], r[8]

        def fold_sum(v):
            sm[pl.ds(0, 16)] = v
            r = v + sm[pl.ds(4, 16)]
            sm[pl.ds(0, 16)] = r
            r = r + sm[pl.ds(2, 16)]
            sm[pl.ds(0, 16)] = r
            r = r + sm[pl.ds(1, 16)]
            return r[0], r[8]

        def process(c, zbuf):
            def group(g, _):
                gbase = g * TG
                zoffs = [(gbase + k) * D_IN for k in range(TG)]

                def dc_body(dc, accs):
                    off = dc * 16
                    wv = [wbuf[pl.ds(e * D_IN + off, 16)]
                          for e in range(N_EXPERTS)]
                    out = []
                    for k in range(TG):
                        zv = zbuf[pl.ds(zoffs[k] + off, 16)]
                        out.append(tuple(accs[k][e] + zv * wv[e]
                                         for e in range(N_EXPERTS)))
                    return tuple(out)

                zero = jnp.zeros((16,), jnp.float32)
                init = tuple(tuple(zero for _ in range(N_EXPERTS))
                             for _ in range(TG))
                accs = lax.fori_loop(0, D_IN // 16, dc_body, init,
                                     unroll=False)

                for k in range(0, TG, 2):
                    v = jnp.zeros((16,), jnp.float32)
                    for kk in (k, k + 1):
                        for e in range(N_EXPERTS):
                            red[pl.ds(e * 16, 16)] = accs[kk][e]
                        for e in range(N_EXPERTS):
                            h = accs[kk][e] + red[pl.ds(e * 16 + 8, 16)]
                            red[pl.ds(e * 16, 16)] = h
                            q = h + red[pl.ds(e * 16 + 4, 16)]
                            red[pl.ds(e * 16, 16)] = q
                            r = q + red[pl.ds(e * 16 + 2, 16)]
                            s = r[0] + r[1]
                            lane = (kk - k) * 8 + e
                            v = jnp.where(iota == lane,
                                          jnp.full((16,), s), v)
                    v = v + bvec
                    m_a, m_b = fold_max(v)
                    m = jnp.where(mask8, jnp.full((16,), m_a),
                                  jnp.full((16,), m_b))
                    e_v = jnp.exp(v - m)
                    s_a, s_b = fold_sum(e_v)
                    d = jnp.where(mask8, jnp.full((16,), s_a),
                                  jnp.full((16,), s_b))
                    p = e_v / d
                    outstage[pl.ds((gbase + k) * N_EXPERTS, 16)] = p
                return 0

            lax.fori_loop(0, T_CH // TG, group, 0, unroll=False)
            tok0 = wbase + c * T_CH
            pltpu.sync_copy(outstage.at[pl.ds(0, T_CH * N_EXPERTS)],
                            out_hbm.at[pl.ds(tok0 * N_EXPERTS,
                                             T_CH * N_EXPERTS)])

        start(0, zbuf0, sem0)
        start(1, zbuf1, sem1)

        def chunk_pair(i, _):
            c0 = 2 * i
            wait(c0, zbuf0, sem0)
            process(c0, zbuf0)

            @pl.when(c0 + 2 < n_chunks)
            def _():
                start(c0 + 2, zbuf0, sem0)

            c1 = c0 + 1
            wait(c1, zbuf1, sem1)
            process(c1, zbuf1)

            @pl.when(c1 + 2 < n_chunks)
            def _():
                start(c1 + 2, zbuf1, sem1)
            return 0

        lax.fori_loop(0, n_chunks // 2, chunk_pair, 0, unroll=False)

    return _sc_body


_SC_SCRATCH = [
    pltpu.VMEM((T_CH * D_IN,), jnp.float32),
    pltpu.VMEM((T_CH * D_IN,), jnp.float32),
    pltpu.VMEM((N_EXPERTS * D_IN,), jnp.float32),
    pltpu.VMEM((16,), jnp.float32),
    pltpu.VMEM((N_EXPERTS * 16 + 16,), jnp.float32),
    pltpu.VMEM((32,), jnp.float32),
    pltpu.VMEM((T_CH * N_EXPERTS,), jnp.float32),
    pltpu.SemaphoreType.DMA,
    pltpu.SemaphoreType.DMA,
]


def _sc_router(z, W, b, n_sc, base):
    zf = z.reshape(z.shape[0] * D_IN)
    wf = W.reshape(N_EXPERTS * D_IN)
    b16 = jnp.concatenate([b, b])
    mesh = plsc.VectorSubcoreMesh(core_axis_name="c", subcore_axis_name="s")
    out = pl.kernel(
        _sc_body_for(n_sc, base),
        out_type=jax.ShapeDtypeStruct((n_sc * N_EXPERTS,), jnp.float32),
        mesh=mesh,
        scratch_types=_SC_SCRATCH,
    )(zf, wf, b16)
    return out.reshape(n_sc, N_EXPERTS)


N_SC = 4096


def kernel(z, W, b):
    n = z.shape[0]
    b2 = b.reshape(1, N_EXPERTS)
    out_tc = _tc_router(z, W, b2, n - N_SC)
    out_sc = _sc_router(z, W, b, N_SC, n - N_SC)
    return jnp.concatenate([out_tc, out_sc], axis=0)


# speedup vs baseline: 5.5584x; 2.2093x over previous
import jax
import jax.numpy as jnp
from jax import lax
from jax.experimental import pallas as pl
from jax.experimental.pallas import tpu as pltpu
from jax.experimental.pallas import tpu_sc as plsc

N_TOKENS = 32768
D_IN = 1024
N_EXPERTS = 8


BT = 2048


def _tc_body(z_ref, w_ref, b_ref, out_ref):
    z = z_ref[...]
    w = w_ref[...]
    logits = lax.dot_general(z, w, (((1,), (1,)), ((), ())),
                             preferred_element_type=jnp.float32)
    logits = logits + b_ref[...]
    m = jnp.max(logits, axis=-1, keepdims=True)
    e = jnp.exp(logits - m)
    s = jnp.sum(e, axis=-1, keepdims=True)
    out_ref[...] = e / s


def _tc_router(z, W, b2, n):
    return pl.pallas_call(
        _tc_body,
        grid=(n // BT,),
        in_specs=[
            pl.BlockSpec((BT, D_IN), lambda i: (i, 0)),
            pl.BlockSpec((N_EXPERTS, D_IN), lambda i: (0, 0)),
            pl.BlockSpec((1, N_EXPERTS), lambda i: (0, 0)),
        ],
        out_specs=pl.BlockSpec((BT, N_EXPERTS), lambda i: (i, 0)),
        out_shape=jax.ShapeDtypeStruct((n, N_EXPERTS), jnp.float32),
    )(z, W, b2)


NW = 32
T_CH = 32
TG = 4


def _sc_body_for(n_sc, base):
    tpw = n_sc // NW
    n_chunks = tpw // T_CH

    def _sc_body(z_hbm, w_hbm, b_hbm, out_hbm,
                 zbuf0, zbuf1, wbuf, bbuf, red, sm, outstage, sem0, sem1):
        wid = lax.axis_index("s") * 2 + lax.axis_index("c")
        wbase = wid * tpw

        pltpu.sync_copy(w_hbm, wbuf)
        pltpu.sync_copy(b_hbm, bbuf)
        bvec = bbuf[...]

        iota = lax.iota(jnp.int32, 16)
        mask8 = iota < 8

        def z_src(c):
            return z_hbm.at[pl.ds(base + wbase + c * T_CH, T_CH), :]

        def start(c, buf, sem):
            pltpu.async_copy(z_src(c), buf, sem)

        def wait(c, buf, sem):
            pltpu.make_async_copy(z_src(c), buf, sem).wait()

        def fold_max(v):
            sm[pl.ds(0, 16)] = v
            r = jnp.maximum(v, sm[pl.ds(4, 16)])
            sm[pl.ds(0, 16)] = r
            r = jnp.maximum(r, sm[pl.ds(2, 16)])
            sm[pl.ds(0, 16)] = r
            r = jnp.maximum(r, sm[pl.ds(1, 16)])
            return r[0], r[8]

        def fold_sum(v):
            sm[pl.ds(0, 16)] = v
            r = v + sm[pl.ds(4, 16)]
            sm[pl.ds(0, 16)] = r
            r = r + sm[pl.ds(2, 16)]
            sm[pl.ds(0, 16)] = r
            r = r + sm[pl.ds(1, 16)]
            return r[0], r[8]

        def process(c, zbuf):
            def group(g, _):
                gbase = g * TG
                toks = [gbase + k for k in range(TG)]

                def dc_body(dc, accs):
                    off = dc * 16
                    wv = [wbuf[pl.ds(e * D_IN + off, 16)]
                          for e in range(N_EXPERTS)]
                    out = []
                    for k in range(TG):
                        zv = zbuf[toks[k], pl.ds(off, 16)]
                        out.append(tuple(accs[k][e] + zv * wv[e]
                                         for e in range(N_EXPERTS)))
                    return tuple(out)

                zero = jnp.zeros((16,), jnp.float32)
                init = tuple(tuple(zero for _ in range(N_EXPERTS))
                             for _ in range(TG))
                accs = lax.fori_loop(0, D_IN // 16, dc_body, init,
                                     unroll=False)

                for k in range(0, TG, 2):
                    v = jnp.zeros((16,), jnp.float32)
                    for kk in (k, k + 1):
                        for e in range(N_EXPERTS):
                            red[pl.ds(e * 16, 16)] = accs[kk][e]
                        for e in range(N_EXPERTS):
                            h = accs[kk][e] + red[pl.ds(e * 16 + 8, 16)]
                            red[pl.ds(e * 16, 16)] = h
                            q = h + red[pl.ds(e * 16 + 4, 16)]
                            red[pl.ds(e * 16, 16)] = q
                            r = q + red[pl.ds(e * 16 + 2, 16)]
                            s = r[0] + r[1]
                            lane = (kk - k) * 8 + e
                            v = jnp.where(iota == lane,
                                          jnp.full((16,), s), v)
                    v = v + bvec
                    m_a, m_b = fold_max(v)
                    m = jnp.where(mask8, jnp.full((16,), m_a),
                                  jnp.full((16,), m_b))
                    e_v = jnp.exp(v - m)
                    s_a, s_b = fold_sum(e_v)
                    d = jnp.where(mask8, jnp.full((16,), s_a),
                                  jnp.full((16,), s_b))
                    p = e_v / d
                    outstage[pl.ds((gbase + k) * N_EXPERTS, 16)] = p
                return 0

            lax.fori_loop(0, T_CH // TG, group, 0, unroll=False)
            tok0 = wbase + c * T_CH
            pltpu.sync_copy(outstage.at[pl.ds(0, T_CH * N_EXPERTS)],
                            out_hbm.at[pl.ds(tok0 * N_EXPERTS,
                                             T_CH * N_EXPERTS)])

        start(0, zbuf0, sem0)
        start(1, zbuf1, sem1)

        def chunk_pair(i, _):
            c0 = 2 * i
            wait(c0, zbuf0, sem0)
            process(c0, zbuf0)

            @pl.when(c0 + 2 < n_chunks)
            def _():
                start(c0 + 2, zbuf0, sem0)

            c1 = c0 + 1
            wait(c1, zbuf1, sem1)
            process(c1, zbuf1)

            @pl.when(c1 + 2 < n_chunks)
            def _():
                start(c1 + 2, zbuf1, sem1)
            return 0

        lax.fori_loop(0, n_chunks // 2, chunk_pair, 0, unroll=False)

    return _sc_body


_SC_SCRATCH = [
    pltpu.VMEM((T_CH, D_IN), jnp.float32),
    pltpu.VMEM((T_CH, D_IN), jnp.float32),
    pltpu.VMEM((N_EXPERTS * D_IN,), jnp.float32),
    pltpu.VMEM((16,), jnp.float32),
    pltpu.VMEM((N_EXPERTS * 16 + 16,), jnp.float32),
    pltpu.VMEM((32,), jnp.float32),
    pltpu.VMEM((T_CH * N_EXPERTS,), jnp.float32),
    pltpu.SemaphoreType.DMA,
    pltpu.SemaphoreType.DMA,
]


def _sc_router(z, W, b, n_sc, base):
    wf = W.reshape(N_EXPERTS * D_IN)
    b16 = jnp.concatenate([b, b])
    mesh = plsc.VectorSubcoreMesh(core_axis_name="c", subcore_axis_name="s")
    out = pl.kernel(
        _sc_body_for(n_sc, base),
        out_type=jax.ShapeDtypeStruct((n_sc * N_EXPERTS,), jnp.float32),
        mesh=mesh,
        scratch_types=_SC_SCRATCH,
    )(z, wf, b16)
    return out.reshape(n_sc, N_EXPERTS)


N_SC = 4096


def kernel(z, W, b):
    n = z.shape[0]
    b2 = b.reshape(1, N_EXPERTS)
    out_tc = _tc_router(z, W, b2, n - N_SC)
    out_sc = _sc_router(z, W, b, N_SC, n - N_SC)
    return jnp.concatenate([out_tc, out_sc], axis=0)


# speedup vs baseline: 5.7957x; 1.0427x over previous
import jax
import jax.numpy as jnp
from jax import lax
from jax.experimental import pallas as pl
from jax.experimental.pallas import tpu as pltpu
from jax.experimental.pallas import tpu_sc as plsc

N_TOKENS = 32768
D_IN = 1024
N_EXPERTS = 8


BT = 2048


def _tc_body(z_ref, w_ref, b_ref, out_ref):
    z = z_ref[...]
    w = w_ref[...]
    logits = lax.dot_general(z, w, (((1,), (1,)), ((), ())),
                             preferred_element_type=jnp.float32)
    logits = logits + b_ref[...]
    m = jnp.max(logits, axis=-1, keepdims=True)
    e = jnp.exp(logits - m)
    s = jnp.sum(e, axis=-1, keepdims=True)
    out_ref[...] = e / s


def _tc_router(z, W, b2, n):
    return pl.pallas_call(
        _tc_body,
        grid=(n // BT,),
        in_specs=[
            pl.BlockSpec((BT, D_IN), lambda i: (i, 0)),
            pl.BlockSpec((N_EXPERTS, D_IN), lambda i: (0, 0)),
            pl.BlockSpec((1, N_EXPERTS), lambda i: (0, 0)),
        ],
        out_specs=pl.BlockSpec((BT, N_EXPERTS), lambda i: (i, 0)),
        out_shape=jax.ShapeDtypeStruct((n, N_EXPERTS), jnp.float32),
    )(z, W, b2)


NW = 32
T_CH = 32
TG = 4


def _sc_body_for(n_sc, base):
    tpw = n_sc // NW
    n_chunks = tpw // T_CH

    def _sc_body(z_hbm, w_hbm, b_hbm, out_hbm,
                 zbuf0, zbuf1, wbuf, bbuf, red, sm, outstage, sem0, sem1):
        wid = lax.axis_index("s") * 2 + lax.axis_index("c")
        wbase = wid * tpw

        pltpu.sync_copy(w_hbm, wbuf)
        pltpu.sync_copy(b_hbm, bbuf)
        bvec = bbuf[...]

        iota = lax.iota(jnp.int32, 16)
        mask8 = iota < 8

        def z_src(c):
            return z_hbm.at[pl.ds(base + wbase + c * T_CH, T_CH), :]

        def start(c, buf, sem):
            pltpu.async_copy(z_src(c), buf, sem)

        def wait(c, buf, sem):
            pltpu.make_async_copy(z_src(c), buf, sem).wait()

        def fold_max(v):
            sm[pl.ds(0, 16)] = v
            r = jnp.maximum(v, sm[pl.ds(4, 16)])
            sm[pl.ds(0, 16)] = r
            r = jnp.maximum(r, sm[pl.ds(2, 16)])
            sm[pl.ds(0, 16)] = r
            r = jnp.maximum(r, sm[pl.ds(1, 16)])
            return r[0], r[8]

        def fold_sum(v):
            sm[pl.ds(0, 16)] = v
            r = v + sm[pl.ds(4, 16)]
            sm[pl.ds(0, 16)] = r
            r = r + sm[pl.ds(2, 16)]
            sm[pl.ds(0, 16)] = r
            r = r + sm[pl.ds(1, 16)]
            return r[0], r[8]

        def process(c, zbuf):
            def group(g, _):
                gbase = g * TG
                toks = [gbase + k for k in range(TG)]

                def dc_body(dc, accs):
                    off = dc * 16
                    wv = [wbuf[pl.ds(e * D_IN + off, 16)]
                          for e in range(N_EXPERTS)]
                    out = []
                    for k in range(TG):
                        zv = zbuf[toks[k], pl.ds(off, 16)]
                        out.append(tuple(accs[k][e] + zv * wv[e]
                                         for e in range(N_EXPERTS)))
                    return tuple(out)

                zero = jnp.zeros((16,), jnp.float32)
                init = tuple(tuple(zero for _ in range(N_EXPERTS))
                             for _ in range(TG))
                accs = lax.fori_loop(0, D_IN // 16, dc_body, init,
                                     unroll=False)

                for k in range(0, TG, 2):
                    v = jnp.zeros((16,), jnp.float32)
                    for kk in (k, k + 1):
                        for e in range(N_EXPERTS):
                            red[pl.ds(e * 16, 16)] = accs[kk][e]
                        for e in range(N_EXPERTS):
                            h = accs[kk][e] + red[pl.ds(e * 16 + 8, 16)]
                            red[pl.ds(e * 16, 16)] = h
                            q = h + red[pl.ds(e * 16 + 4, 16)]
                            red[pl.ds(e * 16, 16)] = q
                            r = q + red[pl.ds(e * 16 + 2, 16)]
                            s = r[0] + r[1]
                            lane = (kk - k) * 8 + e
                            v = jnp.where(iota == lane,
                                          jnp.full((16,), s), v)
                    v = v + bvec
                    m_a, m_b = fold_max(v)
                    m = jnp.where(mask8, jnp.full((16,), m_a),
                                  jnp.full((16,), m_b))
                    e_v = jnp.exp(v - m)
                    s_a, s_b = fold_sum(e_v)
                    d = jnp.where(mask8, jnp.full((16,), s_a),
                                  jnp.full((16,), s_b))
                    p = e_v / d
                    outstage[pl.ds((gbase + k) * N_EXPERTS, 16)] = p
                return 0

            lax.fori_loop(0, T_CH // TG, group, 0, unroll=False)
            tok0 = wbase + c * T_CH
            pltpu.sync_copy(outstage.at[pl.ds(0, T_CH * N_EXPERTS)],
                            out_hbm.at[pl.ds(tok0 * N_EXPERTS,
                                             T_CH * N_EXPERTS)])

        start(0, zbuf0, sem0)
        start(1, zbuf1, sem1)

        def chunk_pair(i, _):
            c0 = 2 * i
            wait(c0, zbuf0, sem0)
            process(c0, zbuf0)

            @pl.when(c0 + 2 < n_chunks)
            def _():
                start(c0 + 2, zbuf0, sem0)

            c1 = c0 + 1
            wait(c1, zbuf1, sem1)
            process(c1, zbuf1)

            @pl.when(c1 + 2 < n_chunks)
            def _():
                start(c1 + 2, zbuf1, sem1)
            return 0

        lax.fori_loop(0, n_chunks // 2, chunk_pair, 0, unroll=False)

    return _sc_body


_SC_SCRATCH = [
    pltpu.VMEM((T_CH, D_IN), jnp.float32),
    pltpu.VMEM((T_CH, D_IN), jnp.float32),
    pltpu.VMEM((N_EXPERTS * D_IN,), jnp.float32),
    pltpu.VMEM((16,), jnp.float32),
    pltpu.VMEM((N_EXPERTS * 16 + 16,), jnp.float32),
    pltpu.VMEM((32,), jnp.float32),
    pltpu.VMEM((T_CH * N_EXPERTS,), jnp.float32),
    pltpu.SemaphoreType.DMA,
    pltpu.SemaphoreType.DMA,
]


def _sc_router(z, W, b, n_sc, base):
    wf = W.reshape(N_EXPERTS * D_IN)
    b16 = jnp.concatenate([b, b])
    mesh = plsc.VectorSubcoreMesh(core_axis_name="c", subcore_axis_name="s")
    out = pl.kernel(
        _sc_body_for(n_sc, base),
        out_type=jax.ShapeDtypeStruct((n_sc * N_EXPERTS,), jnp.float32),
        mesh=mesh,
        scratch_types=_SC_SCRATCH,
    )(z, wf, b16)
    return out.reshape(n_sc, N_EXPERTS)


N_SC = 2048


def kernel(z, W, b):
    n = z.shape[0]
    b2 = b.reshape(1, N_EXPERTS)
    out_sc = _sc_router(z, W, b, N_SC, n - N_SC)
    out_tc = _tc_router(z, W, b2, n - N_SC)
    return jnp.concatenate([out_tc, out_sc], axis=0)
